# Initial kernel scaffold; baseline (speedup 1.0000x reference)
import jax
import jax.numpy as jnp
from jax.experimental import pallas as pl


def kernel(X, nucleo_table, pos_table):
    raise NotImplementedError("write your pallas kernel here")


# speedup vs baseline: 2.0099x; 2.0099x over previous
import functools

import jax
import jax.numpy as jnp
from jax import lax
from jax.experimental import pallas as pl
from jax.experimental.pallas import tpu as pltpu
from jax.experimental.pallas import tpu_sc as plsc

BATCH = 4096
SEQ = 200
DIM = 64
ROWS = BATCH * SEQ
CH = 100
NCHUNKS = ROWS // CH
NC = 2
NS = 16
NW = NC * NS
CPW = NCHUNKS // NW


def _body(xr_hbm, nucleo_hbm, pos_hbm, out_hbm,
          idx_v, rows_v, pos_v, sem):
    wid = lax.axis_index("s") * NC + lax.axis_index("c")
    c0 = wid * CPW

    pltpu.sync_copy(pos_hbm, pos_v)

    def step(i, carry):
        c = c0 + i
        phase = (c % 2) * CH
        pltpu.sync_copy(xr_hbm.at[c], idx_v)
        pltpu.async_copy(nucleo_hbm.at[idx_v], rows_v, sem).wait()

        def add_row(r, carry2):
            pr = phase + r
            for j in range(DIM // 16):
                sl = pl.ds(j * 16, 16)
                rows_v[r, sl] = rows_v[r, sl] + pos_v[pr, sl]
            return carry2

        lax.fori_loop(0, CH, add_row, 0)
        pltpu.sync_copy(rows_v, out_hbm.at[c])
        return carry

    lax.fori_loop(0, CPW, step, 0)


def kernel(X, nucleo_table, pos_table):
    xr = X.reshape(NCHUNKS, CH)
    mesh = plsc.VectorSubcoreMesh(core_axis_name="c", subcore_axis_name="s")
    k = pl.kernel(
        _body,
        mesh=mesh,
        compiler_params=pltpu.CompilerParams(use_tc_tiling_on_sc=False),
        out_type=jax.ShapeDtypeStruct((NCHUNKS, CH, DIM), jnp.float32),
        scratch_types=[
            pltpu.VMEM((CH,), jnp.int32),
            pltpu.VMEM((CH, DIM), jnp.float32),
            pltpu.VMEM((SEQ, DIM), jnp.float32),
            pltpu.SemaphoreType.DMA,
        ],
    )
    out = k(xr, nucleo_table, pos_table)
    return out.reshape(BATCH, SEQ, DIM)


# speedup vs baseline: 3.5580x; 1.7702x over previous
import functools

import jax
import jax.numpy as jnp
from jax import lax
from jax.experimental import pallas as pl
from jax.experimental.pallas import tpu as pltpu
from jax.experimental.pallas import tpu_sc as plsc

BATCH = 4096
SEQ = 200
DIM = 64
ROWS = BATCH * SEQ
CH = 100
NCHUNKS = ROWS // CH
NC = 2
NS = 16
NW = NC * NS
CPW = NCHUNKS // NW


def _body(xr_hbm, nucleo_hbm, pos_hbm, out_hbm,
          idx0, idx1, rows0, rows1, pos_v,
          gsem0, gsem1, ssem0, ssem1):
    idx = (idx0, idx1)
    rows = (rows0, rows1)
    gsem = (gsem0, gsem1)
    ssem = (ssem0, ssem1)
    wid = lax.axis_index("s") * NC + lax.axis_index("c")
    c0 = wid * CPW

    pltpu.sync_copy(pos_hbm, pos_v)

    pltpu.sync_copy(xr_hbm.at[c0], idx[0])
    pltpu.async_copy(nucleo_hbm.at[idx[0]], rows[0], gsem[0])

    def group(g, carry):
        for b in range(2):
            i = g * 2 + b
            s, t = b, 1 - b

            @pl.when(i + 1 < CPW)
            def _stage():
                pltpu.sync_copy(xr_hbm.at[c0 + i + 1], idx[t])

                @pl.when(i >= 1)
                def _drain():
                    pltpu.make_async_copy(
                        rows[t], out_hbm.at[c0 + i - 1], ssem[t]).wait()

                pltpu.async_copy(nucleo_hbm.at[idx[t]], rows[t], gsem[t])

            pltpu.make_async_copy(
                nucleo_hbm.at[idx[s]], rows[s], gsem[s]).wait()

            def add_row(r, carry2):
                for j in range(DIM // 16):
                    sl = pl.ds(j * 16, 16)
                    rows[s][r, sl] = rows[s][r, sl] + pos_v[b * CH + r, sl]
                return carry2

            lax.fori_loop(0, CH, add_row, 0)
            pltpu.async_copy(rows[s], out_hbm.at[c0 + i], ssem[s])
        return carry

    lax.fori_loop(0, CPW // 2, group, 0)

    pltpu.make_async_copy(rows[0], out_hbm.at[c0 + CPW - 2], ssem[0]).wait()
    pltpu.make_async_copy(rows[1], out_hbm.at[c0 + CPW - 1], ssem[1]).wait()


def kernel(X, nucleo_table, pos_table):
    xr = X.reshape(NCHUNKS, CH)
    mesh = plsc.VectorSubcoreMesh(core_axis_name="c", subcore_axis_name="s")
    k = pl.kernel(
        _body,
        mesh=mesh,
        compiler_params=pltpu.CompilerParams(use_tc_tiling_on_sc=False),
        out_type=jax.ShapeDtypeStruct((NCHUNKS, CH, DIM), jnp.float32),
        scratch_types=[
            pltpu.VMEM((CH,), jnp.int32),
            pltpu.VMEM((CH,), jnp.int32),
            pltpu.VMEM((CH, DIM), jnp.float32),
            pltpu.VMEM((CH, DIM), jnp.float32),
            pltpu.VMEM((SEQ, DIM), jnp.float32),
            pltpu.SemaphoreType.DMA,
            pltpu.SemaphoreType.DMA,
            pltpu.SemaphoreType.DMA,
            pltpu.SemaphoreType.DMA,
        ],
    )
    out = k(xr, nucleo_table, pos_table)
    return out.reshape(BATCH, SEQ, DIM)
